# Initial kernel scaffold; baseline (speedup 1.0000x reference)
import jax
import jax.numpy as jnp
from jax.experimental import pallas as pl


def kernel(x, r_double, r_ij, r_jk, neighbors, neighbor_mask, neighbors_j, neighbors_k, triple_mask, d_ijk, f_double, Wd1, bd1, Wd2, bd2, Wt1, bt1, Wt2, bt2, Wi, Wo, bo, Wdense, bdense):
    raise NotImplementedError("write your pallas kernel here")


# speedup vs baseline: 12.4086x; 12.4086x over previous
import functools

import numpy as np
import jax
import jax.numpy as jnp
from jax import lax
from jax.experimental import pallas as pl
from jax.experimental.pallas import tpu as pltpu
from jax.experimental.pallas import tpu_sc as plsc

CUTOFF = 5.0
LOG2 = float(np.log(2.0))


def _ssp(v):
    return jnp.maximum(v, 0.0) + jnp.log1p(jnp.exp(-jnp.abs(v))) - LOG2


def _cos_cut(r):
    return 0.5 * (jnp.cos(r * (np.pi / CUTOFF)) + 1.0) * (r < CUTOFF).astype(r.dtype)


def _in2f_body(x_ref, wi_ref, y_ref):
    y_ref[:] = jnp.dot(x_ref[:], wi_ref[:], preferred_element_type=jnp.float32)


def _in2f(x2, Wi):
    M, K = x2.shape
    F = Wi.shape[1]
    return pl.pallas_call(
        _in2f_body,
        out_shape=jax.ShapeDtypeStruct((M, F), jnp.float32),
    )(x2, Wi)


def _sc_gather_call(y, gd, gj, gk, NW, CH):
    R, F = y.shape
    n_ch = gd.shape[1]
    E = NW * n_ch * CH
    per_w = n_ch * CH
    mesh = plsc.VectorSubcoreMesh(core_axis_name="c", subcore_axis_name="s")
    out_sds = jax.ShapeDtypeStruct((E, F), jnp.float32)

    @functools.partial(
        pl.kernel,
        out_type=[out_sds, out_sds, out_sds],
        mesh=mesh,
        scratch_types=[
            pltpu.VMEM((n_ch, CH), jnp.int32),
            pltpu.VMEM((n_ch, CH), jnp.int32),
            pltpu.VMEM((n_ch, CH), jnp.int32),
            pltpu.VMEM((CH, F), jnp.float32),
            pltpu.VMEM((CH, F), jnp.float32),
            pltpu.VMEM((CH, F), jnp.float32),
            pltpu.SemaphoreType.DMA,
        ],
    )
    def sc_gather(y_hbm, gd_hbm, gj_hbm, gk_hbm, od_hbm, oj_hbm, ok_hbm,
                  idx_d, idx_j, idx_k, rows_d, rows_j, rows_k, sem):
        wid = lax.axis_index("s") * 2 + lax.axis_index("c")
        pltpu.sync_copy(gd_hbm.at[wid], idx_d)
        pltpu.sync_copy(gj_hbm.at[wid], idx_j)
        pltpu.sync_copy(gk_hbm.at[wid], idx_k)
        base = wid * per_w

        def body(i, carry):
            e0 = base + i * CH
            cpd = pltpu.async_copy(y_hbm.at[idx_d.at[i]], rows_d, sem)
            cpj = pltpu.async_copy(y_hbm.at[idx_j.at[i]], rows_j, sem)
            cpk = pltpu.async_copy(y_hbm.at[idx_k.at[i]], rows_k, sem)
            cpd.wait()
            cpj.wait()
            cpk.wait()
            pltpu.sync_copy(rows_d, od_hbm.at[pl.ds(e0, CH)])
            pltpu.sync_copy(rows_j, oj_hbm.at[pl.ds(e0, CH)])
            pltpu.sync_copy(rows_k, ok_hbm.at[pl.ds(e0, CH)])
            return carry

        lax.fori_loop(0, n_ch, body, 0)

    return sc_gather(y, gd, gj, gk)


def _main_body(BA, N, fd_ref, dt_ref, ydg_ref, yj_ref, yk_ref,
               rd_ref, rij_ref, rjk_ref, nm_ref, tm_ref,
               wd1, bd1, wd2, bd2, wt1, bt1, wt2, bt2, wo, bo, wdn, bdn,
               out_ref):
    f32 = jnp.float32
    F = wd2.shape[1]
    Wd = _ssp(jnp.dot(fd_ref[:], wd1[:], preferred_element_type=f32) + bd1[:])
    Wd = _ssp(jnp.dot(Wd, wd2[:], preferred_element_type=f32) + bd2[:])
    Wt = _ssp(jnp.dot(dt_ref[:], wt1[:], preferred_element_type=f32) + bt1[:])
    Wt = _ssp(jnp.dot(Wt, wt2[:], preferred_element_type=f32) + bt2[:])
    cutd = _cos_cut(rd_ref[:]) * nm_ref[:]
    cutt = _cos_cut(rij_ref[:]) * _cos_cut(rjk_ref[:]) * tm_ref[:]
    cd = (ydg_ref[:] * Wd).reshape(BA, N, F) * cutd[:, :, None]
    ct = (yj_ref[:] * yk_ref[:] * Wt).reshape(BA, N, F) * cutt[:, :, None]
    v = jnp.sum(cd + ct, axis=1)
    v = _ssp(jnp.dot(v, wo[:], preferred_element_type=f32) + bo[:])
    out_ref[:] = jnp.dot(v, wdn[:], preferred_element_type=f32) + bdn[:]


def _main(fd2, dt2, ydg, yjg, ykg, rd2, rij2, rjk2, nm2, tm2,
          Wd1, bd1, Wd2, bd2, Wt1, bt1, Wt2, bt2, Wo, bo, Wdense, bdense,
          BA, N):
    M = rd2.shape[0]
    E, F = ydg.shape
    nsp = Wd1.shape[0]
    dtr = Wt1.shape[0]
    EB = BA * N
    grid = (M // BA,)

    def eb(i):
        return (i, 0)

    def full(i):
        return (0, 0)

    edge_spec = lambda K: pl.BlockSpec((EB, K), eb)
    atom_spec = pl.BlockSpec((BA, N), eb)
    w_spec = lambda s: pl.BlockSpec(s, full)

    return pl.pallas_call(
        functools.partial(_main_body, BA, N),
        grid=grid,
        in_specs=[
            edge_spec(nsp), edge_spec(dtr),
            edge_spec(F), edge_spec(F), edge_spec(F),
            atom_spec, atom_spec, atom_spec, atom_spec, atom_spec,
            w_spec((nsp, F)), w_spec((1, F)), w_spec((F, F)), w_spec((1, F)),
            w_spec((dtr, F)), w_spec((1, F)), w_spec((F, F)), w_spec((1, F)),
            w_spec((F, F)), w_spec((1, F)), w_spec((F, F)), w_spec((1, F)),
        ],
        out_specs=pl.BlockSpec((BA, F), eb),
        out_shape=jax.ShapeDtypeStruct((M, F), jnp.float32),
    )(fd2, dt2, ydg, yjg, ykg, rd2, rij2, rjk2, nm2, tm2,
      Wd1, bd1.reshape(1, F), Wd2, bd2.reshape(1, F),
      Wt1, bt1.reshape(1, F), Wt2, bt2.reshape(1, F),
      Wo, bo.reshape(1, F), Wdense, bdense.reshape(1, F))


def kernel(x, r_double, r_ij, r_jk, neighbors, neighbor_mask, neighbors_j,
           neighbors_k, triple_mask, d_ijk, f_double,
           Wd1, bd1, Wd2, bd2, Wt1, bt1, Wt2, bt2, Wi, Wo, bo, Wdense, bdense):
    B, A, N = neighbors.shape
    nb = x.shape[-1]
    nsp = Wd1.shape[0]
    dtr = Wt1.shape[0]
    E = B * A * N
    NW = 32
    CH = 128
    n_ch = E // (NW * CH)

    y = _in2f(x.reshape(B * A, nb), Wi)

    base = (jnp.arange(B, dtype=jnp.int32) * A)[:, None, None]
    shp = (NW, n_ch, CH)
    gd = (neighbors.astype(jnp.int32) + base).reshape(shp)
    gj = (neighbors_j.astype(jnp.int32) + base).reshape(shp)
    gk = (neighbors_k.astype(jnp.int32) + base).reshape(shp)
    ydg, yjg, ykg = _sc_gather_call(y, gd, gj, gk, NW, CH)

    BA = 64
    out = _main(
        f_double.reshape(E, nsp), d_ijk.reshape(E, dtr), ydg, yjg, ykg,
        r_double.reshape(B * A, N), r_ij.reshape(B * A, N),
        r_jk.reshape(B * A, N), neighbor_mask.reshape(B * A, N),
        triple_mask.reshape(B * A, N),
        Wd1, bd1, Wd2, bd2, Wt1, bt1, Wt2, bt2, Wo, bo, Wdense, bdense,
        BA, N)
    return out.reshape(B, A, nb)


# speedup vs baseline: 14.2768x; 1.1506x over previous
import functools

import numpy as np
import jax
import jax.numpy as jnp
from jax import lax
from jax.experimental import pallas as pl
from jax.experimental.pallas import tpu as pltpu
from jax.experimental.pallas import tpu_sc as plsc

CUTOFF = 5.0
LOG2 = float(np.log(2.0))


def _ssp(v):
    sp = jnp.log(1.0 + jnp.exp(jnp.minimum(v, 60.0)))
    return jnp.maximum(sp, v) - LOG2


def _cos_cut(r):
    return 0.5 * (jnp.cos(r * (np.pi / CUTOFF)) + 1.0) * (r < CUTOFF).astype(r.dtype)


def _in2f_body(x_ref, wi_ref, y_ref):
    y_ref[:] = jnp.dot(x_ref[:], wi_ref[:], preferred_element_type=jnp.float32)


def _in2f(x2, Wi):
    M, K = x2.shape
    F = Wi.shape[1]
    return pl.pallas_call(
        _in2f_body,
        out_shape=jax.ShapeDtypeStruct((M, F), jnp.float32),
    )(x2, Wi)


def _sc_gather_call(y, gd, gj, gk, NW, CH):
    R, F = y.shape
    n_ch = gd.shape[1]
    E = NW * n_ch * CH
    per_w = n_ch * CH
    mesh = plsc.VectorSubcoreMesh(core_axis_name="c", subcore_axis_name="s")
    out_sds = jax.ShapeDtypeStruct((E, F), jnp.float32)

    @functools.partial(
        pl.kernel,
        out_type=[out_sds, out_sds, out_sds],
        mesh=mesh,
        scratch_types=[
            pltpu.VMEM((n_ch, CH), jnp.int32),
            pltpu.VMEM((n_ch, CH), jnp.int32),
            pltpu.VMEM((n_ch, CH), jnp.int32),
            pltpu.VMEM((CH, F), jnp.float32),
            pltpu.VMEM((CH, F), jnp.float32),
            pltpu.VMEM((CH, F), jnp.float32),
            pltpu.SemaphoreType.DMA,
        ],
    )
    def sc_gather(y_hbm, gd_hbm, gj_hbm, gk_hbm, od_hbm, oj_hbm, ok_hbm,
                  idx_d, idx_j, idx_k, rows_d, rows_j, rows_k, sem):
        wid = lax.axis_index("s") * 2 + lax.axis_index("c")
        pltpu.sync_copy(gd_hbm.at[wid], idx_d)
        pltpu.sync_copy(gj_hbm.at[wid], idx_j)
        pltpu.sync_copy(gk_hbm.at[wid], idx_k)
        base = wid * per_w

        def body(i, carry):
            e0 = base + i * CH
            cpd = pltpu.async_copy(y_hbm.at[idx_d.at[i]], rows_d, sem)
            cpj = pltpu.async_copy(y_hbm.at[idx_j.at[i]], rows_j, sem)
            cpk = pltpu.async_copy(y_hbm.at[idx_k.at[i]], rows_k, sem)
            cpd.wait()
            cpj.wait()
            cpk.wait()
            pltpu.sync_copy(rows_d, od_hbm.at[pl.ds(e0, CH)])
            pltpu.sync_copy(rows_j, oj_hbm.at[pl.ds(e0, CH)])
            pltpu.sync_copy(rows_k, ok_hbm.at[pl.ds(e0, CH)])
            return carry

        lax.fori_loop(0, n_ch, body, 0)

    return sc_gather(y, gd, gj, gk)


def _main_body(BA, N, fd_ref, dt_ref, ydg_ref, yj_ref, yk_ref,
               rd_ref, rij_ref, rjk_ref, nm_ref, tm_ref,
               wd1, bd1, wd2, bd2, wt1, bt1, wt2, bt2, wo, bo, wdn, bdn,
               out_ref):
    f32 = jnp.float32
    F = wd2.shape[1]
    Wd = _ssp(jnp.dot(fd_ref[:], wd1[:], preferred_element_type=f32) + bd1[:])
    Wd = _ssp(jnp.dot(Wd, wd2[:], preferred_element_type=f32) + bd2[:])
    Wt = _ssp(jnp.dot(dt_ref[:], wt1[:], preferred_element_type=f32) + bt1[:])
    Wt = _ssp(jnp.dot(Wt, wt2[:], preferred_element_type=f32) + bt2[:])
    cutd = _cos_cut(rd_ref[:]) * nm_ref[:]
    cutt = _cos_cut(rij_ref[:]) * _cos_cut(rjk_ref[:]) * tm_ref[:]
    cd = (ydg_ref[:] * Wd).reshape(BA, N, F) * cutd[:, :, None]
    ct = (yj_ref[:] * yk_ref[:] * Wt).reshape(BA, N, F) * cutt[:, :, None]
    v = jnp.sum(cd + ct, axis=1)
    v = _ssp(jnp.dot(v, wo[:], preferred_element_type=f32) + bo[:])
    out_ref[:] = jnp.dot(v, wdn[:], preferred_element_type=f32) + bdn[:]


def _main(fd2, dt2, ydg, yjg, ykg, rd2, rij2, rjk2, nm2, tm2,
          Wd1, bd1, Wd2, bd2, Wt1, bt1, Wt2, bt2, Wo, bo, Wdense, bdense,
          BA, N):
    M = rd2.shape[0]
    E, F = ydg.shape
    nsp = Wd1.shape[0]
    dtr = Wt1.shape[0]
    EB = BA * N
    grid = (M // BA,)

    def eb(i):
        return (i, 0)

    def full(i):
        return (0, 0)

    edge_spec = lambda K: pl.BlockSpec((EB, K), eb)
    atom_spec = pl.BlockSpec((BA, N), eb)
    w_spec = lambda s: pl.BlockSpec(s, full)

    return pl.pallas_call(
        functools.partial(_main_body, BA, N),
        grid=grid,
        in_specs=[
            edge_spec(nsp), edge_spec(dtr),
            edge_spec(F), edge_spec(F), edge_spec(F),
            atom_spec, atom_spec, atom_spec, atom_spec, atom_spec,
            w_spec((nsp, F)), w_spec((1, F)), w_spec((F, F)), w_spec((1, F)),
            w_spec((dtr, F)), w_spec((1, F)), w_spec((F, F)), w_spec((1, F)),
            w_spec((F, F)), w_spec((1, F)), w_spec((F, F)), w_spec((1, F)),
        ],
        out_specs=pl.BlockSpec((BA, F), eb),
        out_shape=jax.ShapeDtypeStruct((M, F), jnp.float32),
    )(fd2, dt2, ydg, yjg, ykg, rd2, rij2, rjk2, nm2, tm2,
      Wd1, bd1.reshape(1, F), Wd2, bd2.reshape(1, F),
      Wt1, bt1.reshape(1, F), Wt2, bt2.reshape(1, F),
      Wo, bo.reshape(1, F), Wdense, bdense.reshape(1, F))


def kernel(x, r_double, r_ij, r_jk, neighbors, neighbor_mask, neighbors_j,
           neighbors_k, triple_mask, d_ijk, f_double,
           Wd1, bd1, Wd2, bd2, Wt1, bt1, Wt2, bt2, Wi, Wo, bo, Wdense, bdense):
    B, A, N = neighbors.shape
    nb = x.shape[-1]
    nsp = Wd1.shape[0]
    dtr = Wt1.shape[0]
    E = B * A * N
    NW = 32
    CH = 128
    n_ch = E // (NW * CH)

    y = _in2f(x.reshape(B * A, nb), Wi)

    base = (jnp.arange(B, dtype=jnp.int32) * A)[:, None, None]
    shp = (NW, n_ch, CH)
    gd = (neighbors.astype(jnp.int32) + base).reshape(shp)
    gj = (neighbors_j.astype(jnp.int32) + base).reshape(shp)
    gk = (neighbors_k.astype(jnp.int32) + base).reshape(shp)
    ydg, yjg, ykg = _sc_gather_call(y, gd, gj, gk, NW, CH)

    BA = 64
    out = _main(
        f_double.reshape(E, nsp), d_ijk.reshape(E, dtr), ydg, yjg, ykg,
        r_double.reshape(B * A, N), r_ij.reshape(B * A, N),
        r_jk.reshape(B * A, N), neighbor_mask.reshape(B * A, N),
        triple_mask.reshape(B * A, N),
        Wd1, bd1, Wd2, bd2, Wt1, bt1, Wt2, bt2, Wo, bo, Wdense, bdense,
        BA, N)
    return out.reshape(B, A, nb)


# speedup vs baseline: 15.3491x; 1.0751x over previous
import functools

import numpy as np
import jax
import jax.numpy as jnp
from jax import lax
from jax.experimental import pallas as pl
from jax.experimental.pallas import tpu as pltpu
from jax.experimental.pallas import tpu_sc as plsc

CUTOFF = 5.0
LOG2 = float(np.log(2.0))


def _ssp(v):
    sp = jnp.log(1.0 + jnp.exp(jnp.minimum(v, 60.0)))
    return jnp.maximum(sp, v) - LOG2


def _cos_cut(r):
    return 0.5 * (jnp.cos(r * (np.pi / CUTOFF)) + 1.0) * (r < CUTOFF).astype(r.dtype)


def _in2f_body(x_ref, wi_ref, y_ref):
    y_ref[:] = jnp.dot(x_ref[:], wi_ref[:], preferred_element_type=jnp.float32)


def _in2f(x2, Wi):
    M, K = x2.shape
    F = Wi.shape[1]
    return pl.pallas_call(
        _in2f_body,
        out_shape=jax.ShapeDtypeStruct((M, F), jnp.float32),
    )(x2, Wi)


def _sc_gather_call(y, gd, gj, gk, NW, CH):
    R, F = y.shape
    n_ch = gd.shape[1]
    E = NW * n_ch * CH
    per_w = n_ch * CH
    mesh = plsc.VectorSubcoreMesh(core_axis_name="c", subcore_axis_name="s")
    out_sds = jax.ShapeDtypeStruct((E, F), jnp.float32)

    rows_t = pltpu.VMEM((CH, F), jnp.float32)
    idx_t = pltpu.VMEM((n_ch, CH), jnp.int32)

    @functools.partial(
        pl.kernel,
        out_type=[out_sds, out_sds, out_sds],
        mesh=mesh,
        scratch_types=[
            idx_t, idx_t, idx_t,
            rows_t, rows_t, rows_t, rows_t, rows_t, rows_t,
            pltpu.SemaphoreType.DMA,
            pltpu.SemaphoreType.DMA,
        ],
    )
    def sc_gather(y_hbm, gd_hbm, gj_hbm, gk_hbm, od_hbm, oj_hbm, ok_hbm,
                  idx_d, idx_j, idx_k, rd0, rj0, rk0, rd1, rj1, rk1,
                  sem_g, sem_w):
        wid = lax.axis_index("s") * 2 + lax.axis_index("c")
        pltpu.sync_copy(gd_hbm.at[wid], idx_d)
        pltpu.sync_copy(gj_hbm.at[wid], idx_j)
        pltpu.sync_copy(gk_hbm.at[wid], idx_k)
        base = wid * per_w
        idxs = (idx_d, idx_j, idx_k)
        bufs = ((rd0, rj0, rk0), (rd1, rj1, rk1))
        outs = (od_hbm, oj_hbm, ok_hbm)

        def phase(it, i, rows):
            @pl.when(it >= 1)
            def _():
                for t in range(3):
                    pltpu.make_async_copy(
                        rows[t], outs[t].at[pl.ds(0, CH)], sem_w).wait()
            cps = [pltpu.async_copy(y_hbm.at[idxs[t].at[i]], rows[t], sem_g)
                   for t in range(3)]
            for cp in cps:
                cp.wait()
            e0 = base + i * CH
            for t in range(3):
                pltpu.async_copy(rows[t], outs[t].at[pl.ds(e0, CH)], sem_w)

        def body(it, carry):
            phase(it, 2 * it, bufs[0])
            phase(it, 2 * it + 1, bufs[1])
            return carry

        lax.fori_loop(0, n_ch // 2, body, 0)
        for p in range(2):
            for t in range(3):
                pltpu.make_async_copy(
                    bufs[p][t], outs[t].at[pl.ds(0, CH)], sem_w).wait()

    return sc_gather(y, gd, gj, gk)


def _main_body(BA, N, fd_ref, dt_ref, ydg_ref, yj_ref, yk_ref,
               rd_ref, rij_ref, rjk_ref, nm_ref, tm_ref,
               wd1, bd1, wd2, bd2, wt1, bt1, wt2, bt2, wo, bo, wdn, bdn,
               out_ref):
    f32 = jnp.float32
    bf16 = jnp.bfloat16
    F = wd2.shape[1]


    Wd = _ssp(jnp.dot(fd_ref[:], wd1[:], preferred_element_type=f32) + bd1[:])
    Wd = _ssp(jnp.dot(Wd.astype(bf16), wd2[:], preferred_element_type=f32)
              + bd2[:])
    Wt = _ssp(jnp.dot(dt_ref[:], wt1[:], preferred_element_type=f32) + bt1[:])
    Wt = _ssp(jnp.dot(Wt.astype(bf16), wt2[:], preferred_element_type=f32)
              + bt2[:])
    cutd = _cos_cut(rd_ref[:]) * nm_ref[:]
    cutt = _cos_cut(rij_ref[:]) * _cos_cut(rjk_ref[:]) * tm_ref[:]
    cd = (ydg_ref[:] * Wd).reshape(BA, N, F) * cutd[:, :, None]
    ct = (yj_ref[:] * yk_ref[:] * Wt).reshape(BA, N, F) * cutt[:, :, None]
    v = jnp.sum(cd + ct, axis=1)
    v = _ssp(jnp.dot(v, wo[:], preferred_element_type=f32) + bo[:])
    out_ref[:] = jnp.dot(v, wdn[:], preferred_element_type=f32) + bdn[:]


def _main(fd2, dt2, ydg, yjg, ykg, rd2, rij2, rjk2, nm2, tm2,
          Wd1, bd1, Wd2, bd2, Wt1, bt1, Wt2, bt2, Wo, bo, Wdense, bdense,
          BA, N):
    M = rd2.shape[0]
    F = Wo.shape[0]
    nsp = Wd1.shape[0]
    dtr = Wt1.shape[0]
    EB = BA * N
    grid = (M // BA,)

    def eb(i):
        return (i, 0)

    def full(i):
        return (0, 0)

    edge_spec = lambda K: pl.BlockSpec((EB, K), eb)
    atom_spec = pl.BlockSpec((BA, N), eb)
    w_spec = lambda s: pl.BlockSpec(s, full)

    return pl.pallas_call(
        functools.partial(_main_body, BA, N),
        grid=grid,
        in_specs=[
            edge_spec(nsp), edge_spec(dtr),
            edge_spec(F), edge_spec(F), edge_spec(F),
            atom_spec, atom_spec, atom_spec, atom_spec, atom_spec,
            w_spec((nsp, F)), w_spec((1, F)), w_spec((F, F)), w_spec((1, F)),
            w_spec((dtr, F)), w_spec((1, F)), w_spec((F, F)), w_spec((1, F)),
            w_spec((F, F)), w_spec((1, F)), w_spec((F, F)), w_spec((1, F)),
        ],
        out_specs=pl.BlockSpec((BA, F), eb),
        out_shape=jax.ShapeDtypeStruct((M, F), jnp.float32),
    )(fd2, dt2, ydg, yjg, ykg, rd2, rij2, rjk2, nm2, tm2,
      Wd1.astype(jnp.bfloat16), bd1.reshape(1, F),
      Wd2.astype(jnp.bfloat16), bd2.reshape(1, F),
      Wt1.astype(jnp.bfloat16), bt1.reshape(1, F),
      Wt2.astype(jnp.bfloat16), bt2.reshape(1, F),
      Wo, bo.reshape(1, F), Wdense, bdense.reshape(1, F))


def kernel(x, r_double, r_ij, r_jk, neighbors, neighbor_mask, neighbors_j,
           neighbors_k, triple_mask, d_ijk, f_double,
           Wd1, bd1, Wd2, bd2, Wt1, bt1, Wt2, bt2, Wi, Wo, bo, Wdense, bdense):
    B, A, N = neighbors.shape
    nb = x.shape[-1]
    nsp = Wd1.shape[0]
    dtr = Wt1.shape[0]
    E = B * A * N
    NW = 32
    CH = 128
    n_ch = E // (NW * CH)

    y = _in2f(x.reshape(B * A, nb), Wi)

    base = (jnp.arange(B, dtype=jnp.int32) * A)[:, None, None]
    shp = (NW, n_ch, CH)
    gd = (neighbors.astype(jnp.int32) + base).reshape(shp)
    gj = (neighbors_j.astype(jnp.int32) + base).reshape(shp)
    gk = (neighbors_k.astype(jnp.int32) + base).reshape(shp)
    ydg, yjg, ykg = _sc_gather_call(y, gd, gj, gk, NW, CH)

    BA = 64
    out = _main(
        f_double.reshape(E, nsp).astype(jnp.bfloat16),
        d_ijk.reshape(E, dtr).astype(jnp.bfloat16), ydg, yjg, ykg,
        r_double.reshape(B * A, N), r_ij.reshape(B * A, N),
        r_jk.reshape(B * A, N), neighbor_mask.reshape(B * A, N),
        triple_mask.reshape(B * A, N),
        Wd1, bd1, Wd2, bd2, Wt1, bt1, Wt2, bt2, Wo, bo, Wdense, bdense,
        BA, N)
    return out.reshape(B, A, nb)
